# baseline (speedup 1.0000x reference)
import functools

import jax
import jax.numpy as jnp
from jax import lax
from jax.experimental import pallas as pl
from jax.experimental.pallas import tpu as pltpu
from jax.experimental.pallas import tpu_sc as plsc

_B = 4
_S = 4096
_D = 1024
_NC = 2
_NS = 16
_NW = _NC * _NS
_ROWS_PER_W = _S // _NW
_CH = 32
_NCHUNK = _ROWS_PER_W // _CH

_mesh = plsc.VectorSubcoreMesh(core_axis_name="c", subcore_axis_name="s")


_ROWS_PER_CORE = _S // _NC
_NISS = 8
_ICH = 64
_IROWS = _ROWS_PER_CORE // _NISS
_INCHUNK = _IROWS // _ICH


@functools.partial(
    pl.kernel,
    mesh=_mesh,
    out_type=jax.ShapeDtypeStruct((_B, _S, _D), jnp.float32),
    scratch_types=[
        pltpu.VMEM_SHARED((_NISS, 2, _ICH, _D), jnp.float32),
        pltpu.SemaphoreType.DMA((2,)),
        pltpu.SemaphoreType.DMA((2,)),
    ],
)
def _pe_spmem_only(table_hbm, out_hbm, sbuf, lsem, ssem):
    c = lax.axis_index("c")
    s = lax.axis_index("s")
    core_base = c * _ROWS_PER_CORE

    @pl.when(s < _NISS)
    def _issue():
        base = core_base + s * _IROWS

        def load(i):
            return pltpu.async_copy(
                table_hbm.at[pl.ds(base + i * _ICH, _ICH)],
                sbuf.at[s, i % 2], lsem.at[i % 2])

        def store(i, b):
            return pltpu.async_copy(
                sbuf.at[s, i % 2],
                out_hbm.at[b, pl.ds(base + i * _ICH, _ICH)],
                ssem.at[i % 2])

        lh = [None] * _INCHUNK
        sh = [None] * _INCHUNK
        lh[0], lh[1] = load(0), load(1)
        for i in range(_INCHUNK):
            if i > 0 and i + 1 < _INCHUNK:
                for h in sh[i - 1]:
                    h.wait()
                lh[i + 1] = load(i + 1)
            lh[i].wait()
            sh[i] = [store(i, b) for b in range(_B)]
        for i in range(max(0, _INCHUNK - 2), _INCHUNK):
            for h in sh[i]:
                h.wait()


_TROWS = 64
_TCH = 32
_SROWS = _ROWS_PER_CORE - _NS * _TROWS
_SCH = 512


@functools.partial(
    pl.kernel,
    mesh=_mesh,
    out_type=jax.ShapeDtypeStruct((_B, _S, _D), jnp.float32),
    scratch_types=[
        pltpu.VMEM((2, _TCH, _D), jnp.float32),
        pltpu.SemaphoreType.DMA((2,)),
        pltpu.SemaphoreType.DMA((2,)),
        pltpu.VMEM_SHARED((2, _SCH, _D), jnp.float32),
        pltpu.SemaphoreType.DMA((2,)),
        pltpu.SemaphoreType.DMA((2,)),
    ],
)
def _pe_dual(table_hbm, out_hbm, buf, lsem, ssem, sbuf, slsem, sssem):
    c = lax.axis_index("c")
    s = lax.axis_index("s")
    core_base = c * _ROWS_PER_CORE
    tile_base = core_base + s * _TROWS

    def tload(i):
        return pltpu.async_copy(
            table_hbm.at[pl.ds(tile_base + i * _TCH, _TCH)],
            buf.at[i], lsem.at[i])

    def tstore(i, b):
        return pltpu.async_copy(
            buf.at[i],
            out_hbm.at[b, pl.ds(tile_base + i * _TCH, _TCH)],
            ssem.at[i])

    lh = [tload(0), tload(1)]
    sh = []
    for i in range(2):
        lh[i].wait()
        sh += [tstore(i, b) for b in range(_B)]

    spmem_base = core_base + _NS * _TROWS

    @pl.when(s == 0)
    def _spmem_path():
        def sload(i):
            return pltpu.async_copy(
                table_hbm.at[pl.ds(spmem_base + i * _SCH, _SCH)],
                sbuf.at[i], slsem.at[i])

        def sstore(i, b):
            return pltpu.async_copy(
                sbuf.at[i],
                out_hbm.at[b, pl.ds(spmem_base + i * _SCH, _SCH)],
                sssem.at[i])

        slh = [sload(0), sload(1)]
        ssh = []
        for i in range(2):
            slh[i].wait()
            ssh += [sstore(i, b) for b in range(_B)]
        for h in ssh:
            h.wait()

    for h in sh:
        h.wait()


_CH3 = 32
_NSLOT = 3
_NCHUNK3 = _ROWS_PER_W // _CH3


@functools.partial(
    pl.kernel,
    mesh=_mesh,
    out_type=jax.ShapeDtypeStruct((_B, _S, _D), jnp.float32),
    scratch_types=[
        pltpu.VMEM((_NSLOT, _CH3, _D), jnp.float32),
        pltpu.SemaphoreType.DMA((_NSLOT,)),
        pltpu.SemaphoreType.DMA((_NSLOT,)),
    ],
)
def _pe_ring(table_hbm, out_hbm, buf, load_sem, store_sem):
    wid = lax.axis_index("s") * _NC + lax.axis_index("c")
    base = wid * _ROWS_PER_W

    def load(i):
        return pltpu.async_copy(
            table_hbm.at[pl.ds(base + i * _CH3, _CH3)],
            buf.at[i % _NSLOT],
            load_sem.at[i % _NSLOT])

    def store(i, b):
        return pltpu.async_copy(
            buf.at[i % _NSLOT],
            out_hbm.at[b, pl.ds(base + i * _CH3, _CH3)],
            store_sem.at[i % _NSLOT])

    load_h = [None] * _NCHUNK3
    store_h = [None] * _NCHUNK3
    for j in range(min(_NSLOT, _NCHUNK3)):
        load_h[j] = load(j)
    for i in range(_NCHUNK3):
        if i > 0 and i - 1 + _NSLOT < _NCHUNK3:
            for sh in store_h[i - 1]:
                sh.wait()
            load_h[i - 1 + _NSLOT] = load(i - 1 + _NSLOT)
        load_h[i].wait()
        store_h[i] = [store(i, b) for b in range(_B)]
    waited = set(range(0, max(0, _NCHUNK3 - _NSLOT)))
    for i in range(_NCHUNK3):
        if i not in waited:
            for sh in store_h[i]:
                sh.wait()


@functools.partial(
    pl.kernel,
    mesh=_mesh,
    out_type=jax.ShapeDtypeStruct((_B, _S, _D), jnp.float32),
    scratch_types=[
        pltpu.VMEM((2, _CH, _D), jnp.float32),
        pltpu.SemaphoreType.DMA((2,)),
        pltpu.SemaphoreType.DMA((2,)),
    ],
)
def _pe_broadcast(table_hbm, out_hbm, buf, load_sem, store_sem):
    wid = lax.axis_index("s") * _NC + lax.axis_index("c")
    base = wid * _ROWS_PER_W

    def load(i, slot):
        return pltpu.async_copy(
            table_hbm.at[pl.ds(base + i * _CH, _CH)],
            buf.at[slot],
            load_sem.at[slot])

    def store(i, slot, b):
        return pltpu.async_copy(
            buf.at[slot],
            out_hbm.at[b, pl.ds(base + i * _CH, _CH)],
            store_sem.at[slot])

    pending_stores = {0: [], 1: []}
    h = load(0, 0)
    for i in range(_NCHUNK):
        s = i % 2
        if i + 1 < _NCHUNK:
            ns = 1 - s
            for sh in pending_stores[ns]:
                sh.wait()
            pending_stores[ns] = []
            next_h = load(i + 1, ns)
        h.wait()
        pending_stores[s] = [store(i, s, b) for b in range(_B)]
        if i + 1 < _NCHUNK:
            h = next_h
    for s in (0, 1):
        for sh in pending_stores[s]:
            sh.wait()


def kernel(inputs, position_embeddings):
    del inputs
    return _pe_spmem_only(position_embeddings)


# speedup vs baseline: 1.2009x; 1.2009x over previous
import functools

import jax
import jax.numpy as jnp
from jax import lax
from jax.experimental import pallas as pl
from jax.experimental.pallas import tpu as pltpu
from jax.experimental.pallas import tpu_sc as plsc

_B = 4
_S = 4096
_D = 1024
_NC = 2
_NS = 16
_NW = _NC * _NS
_ROWS_PER_W = _S // _NW
_CH = 32
_NCHUNK = _ROWS_PER_W // _CH

_mesh = plsc.VectorSubcoreMesh(core_axis_name="c", subcore_axis_name="s")


_ROWS_PER_CORE = _S // _NC
_NISS = 8
_ICH = 64
_IROWS = _ROWS_PER_CORE // _NISS
_INCHUNK = _IROWS // _ICH


@functools.partial(
    pl.kernel,
    mesh=_mesh,
    out_type=jax.ShapeDtypeStruct((_B, _S, _D), jnp.float32),
    scratch_types=[
        pltpu.VMEM_SHARED((_NISS, 2, _ICH, _D), jnp.float32),
        pltpu.SemaphoreType.DMA((2,)),
        pltpu.SemaphoreType.DMA((2,)),
    ],
)
def _pe_spmem_only(table_hbm, out_hbm, sbuf, lsem, ssem):
    c = lax.axis_index("c")
    s = lax.axis_index("s")
    core_base = c * _ROWS_PER_CORE

    @pl.when(s < _NISS)
    def _issue():
        base = core_base + s * _IROWS

        def load(i):
            return pltpu.async_copy(
                table_hbm.at[pl.ds(base + i * _ICH, _ICH)],
                sbuf.at[s, i % 2], lsem.at[i % 2])

        def store(i, b):
            return pltpu.async_copy(
                sbuf.at[s, i % 2],
                out_hbm.at[b, pl.ds(base + i * _ICH, _ICH)],
                ssem.at[i % 2])

        lh = [None] * _INCHUNK
        sh = [None] * _INCHUNK
        lh[0], lh[1] = load(0), load(1)
        for i in range(_INCHUNK):
            if i > 0 and i + 1 < _INCHUNK:
                for h in sh[i - 1]:
                    h.wait()
                lh[i + 1] = load(i + 1)
            lh[i].wait()
            sh[i] = [store(i, b) for b in range(_B)]
        for i in range(max(0, _INCHUNK - 2), _INCHUNK):
            for h in sh[i]:
                h.wait()


_TROWS = 64
_TCH = 32
_SROWS = _ROWS_PER_CORE - _NS * _TROWS
_SCH = 512


@functools.partial(
    pl.kernel,
    mesh=_mesh,
    out_type=jax.ShapeDtypeStruct((_B, _S, _D), jnp.float32),
    scratch_types=[
        pltpu.VMEM((2, _TCH, _D), jnp.float32),
        pltpu.SemaphoreType.DMA((2,)),
        pltpu.SemaphoreType.DMA((2,)),
        pltpu.VMEM_SHARED((2, _SCH, _D), jnp.float32),
        pltpu.SemaphoreType.DMA((2,)),
        pltpu.SemaphoreType.DMA((2,)),
    ],
)
def _pe_dual(table_hbm, out_hbm, buf, lsem, ssem, sbuf, slsem, sssem):
    c = lax.axis_index("c")
    s = lax.axis_index("s")
    core_base = c * _ROWS_PER_CORE
    tile_base = core_base + s * _TROWS

    def tload(i):
        return pltpu.async_copy(
            table_hbm.at[pl.ds(tile_base + i * _TCH, _TCH)],
            buf.at[i], lsem.at[i])

    def tstore(i, b):
        return pltpu.async_copy(
            buf.at[i],
            out_hbm.at[b, pl.ds(tile_base + i * _TCH, _TCH)],
            ssem.at[i])

    lh = [tload(0), tload(1)]
    sh = []
    for i in range(2):
        lh[i].wait()
        sh += [tstore(i, b) for b in range(_B)]

    spmem_base = core_base + _NS * _TROWS

    @pl.when(s == 0)
    def _spmem_path():
        def sload(i):
            return pltpu.async_copy(
                table_hbm.at[pl.ds(spmem_base + i * _SCH, _SCH)],
                sbuf.at[i], slsem.at[i])

        def sstore(i, b):
            return pltpu.async_copy(
                sbuf.at[i],
                out_hbm.at[b, pl.ds(spmem_base + i * _SCH, _SCH)],
                sssem.at[i])

        slh = [sload(0), sload(1)]
        ssh = []
        for i in range(2):
            slh[i].wait()
            ssh += [sstore(i, b) for b in range(_B)]
        for h in ssh:
            h.wait()

    for h in sh:
        h.wait()


_D_TROWS = 64
_D_TCH = 32
_D_NISS = 8
_D_IROWS = 128
_D_SCH = 64
assert 16 * _D_TROWS + _D_NISS * _D_IROWS == _ROWS_PER_CORE


@functools.partial(
    pl.kernel,
    mesh=_mesh,
    out_type=jax.ShapeDtypeStruct((_B, _S, _D), jnp.float32),
    scratch_types=[
        pltpu.VMEM((2, _D_TCH, _D), jnp.float32),
        pltpu.SemaphoreType.DMA((2,)),
        pltpu.SemaphoreType.DMA((2,)),
        pltpu.VMEM_SHARED((_D_NISS, 2, _D_SCH, _D), jnp.float32),
        pltpu.SemaphoreType.DMA((2,)),
        pltpu.SemaphoreType.DMA((2,)),
    ],
)
def _pe_dual2(table_hbm, out_hbm, tbuf, tlsem, tssem, sbuf, slsem, sssem):
    c = lax.axis_index("c")
    s = lax.axis_index("s")
    core_base = c * _ROWS_PER_CORE
    tile_base = core_base + s * _D_TROWS
    spmem_base = core_base + 16 * _D_TROWS

    def sload_desc(i):
        ibase = spmem_base + s * _D_IROWS
        return pltpu.make_async_copy(
            table_hbm.at[pl.ds(ibase + i * _D_SCH, _D_SCH)],
            sbuf.at[s, i], slsem.at[i])

    @pl.when(s < _D_NISS)
    def _prime():
        sload_desc(0).start()
        sload_desc(1).start()

    tl = [
        pltpu.async_copy(
            table_hbm.at[pl.ds(tile_base + i * _D_TCH, _D_TCH)],
            tbuf.at[i], tlsem.at[i])
        for i in range(2)
    ]
    th = []
    for i in range(2):
        tl[i].wait()
        th += [
            pltpu.async_copy(
                tbuf.at[i],
                out_hbm.at[b, pl.ds(tile_base + i * _D_TCH, _D_TCH)],
                tssem.at[i])
            for b in range(_B)
        ]

    @pl.when(s < _D_NISS)
    def _spmem_stores():
        ibase = spmem_base + s * _D_IROWS
        sh = []
        for i in range(2):
            sload_desc(i).wait()
            sh += [
                pltpu.async_copy(
                    sbuf.at[s, i],
                    out_hbm.at[b, pl.ds(ibase + i * _D_SCH, _D_SCH)],
                    sssem.at[i])
                for b in range(_B)
            ]
        for h in sh:
            h.wait()

    for h in th:
        h.wait()


_CH3 = 32
_NSLOT = 3
_NCHUNK3 = _ROWS_PER_W // _CH3


@functools.partial(
    pl.kernel,
    mesh=_mesh,
    out_type=jax.ShapeDtypeStruct((_B, _S, _D), jnp.float32),
    scratch_types=[
        pltpu.VMEM((_NSLOT, _CH3, _D), jnp.float32),
        pltpu.SemaphoreType.DMA((_NSLOT,)),
        pltpu.SemaphoreType.DMA((_NSLOT,)),
    ],
)
def _pe_ring(table_hbm, out_hbm, buf, load_sem, store_sem):
    wid = lax.axis_index("s") * _NC + lax.axis_index("c")
    base = wid * _ROWS_PER_W

    def load(i):
        return pltpu.async_copy(
            table_hbm.at[pl.ds(base + i * _CH3, _CH3)],
            buf.at[i % _NSLOT],
            load_sem.at[i % _NSLOT])

    def store(i, b):
        return pltpu.async_copy(
            buf.at[i % _NSLOT],
            out_hbm.at[b, pl.ds(base + i * _CH3, _CH3)],
            store_sem.at[i % _NSLOT])

    load_h = [None] * _NCHUNK3
    store_h = [None] * _NCHUNK3
    for j in range(min(_NSLOT, _NCHUNK3)):
        load_h[j] = load(j)
    for i in range(_NCHUNK3):
        if i > 0 and i - 1 + _NSLOT < _NCHUNK3:
            for sh in store_h[i - 1]:
                sh.wait()
            load_h[i - 1 + _NSLOT] = load(i - 1 + _NSLOT)
        load_h[i].wait()
        store_h[i] = [store(i, b) for b in range(_B)]
    waited = set(range(0, max(0, _NCHUNK3 - _NSLOT)))
    for i in range(_NCHUNK3):
        if i not in waited:
            for sh in store_h[i]:
                sh.wait()


@functools.partial(
    pl.kernel,
    mesh=_mesh,
    out_type=jax.ShapeDtypeStruct((_B, _S, _D), jnp.float32),
    scratch_types=[
        pltpu.VMEM((2, _CH, _D), jnp.float32),
        pltpu.SemaphoreType.DMA((2,)),
        pltpu.SemaphoreType.DMA((2,)),
    ],
)
def _pe_broadcast(table_hbm, out_hbm, buf, load_sem, store_sem):
    wid = lax.axis_index("s") * _NC + lax.axis_index("c")
    base = wid * _ROWS_PER_W

    def load(i, slot):
        return pltpu.async_copy(
            table_hbm.at[pl.ds(base + i * _CH, _CH)],
            buf.at[slot],
            load_sem.at[slot])

    def store(i, slot, b):
        return pltpu.async_copy(
            buf.at[slot],
            out_hbm.at[b, pl.ds(base + i * _CH, _CH)],
            store_sem.at[slot])

    pending_stores = {0: [], 1: []}
    h = load(0, 0)
    for i in range(_NCHUNK):
        s = i % 2
        if i + 1 < _NCHUNK:
            ns = 1 - s
            for sh in pending_stores[ns]:
                sh.wait()
            pending_stores[ns] = []
            next_h = load(i + 1, ns)
        h.wait()
        pending_stores[s] = [store(i, s, b) for b in range(_B)]
        if i + 1 < _NCHUNK:
            h = next_h
    for s in (0, 1):
        for sh in pending_stores[s]:
            sh.wait()


def kernel(inputs, position_embeddings):
    del inputs
    return _pe_dual2(position_embeddings)


# speedup vs baseline: 1.2096x; 1.0073x over previous
import functools

import jax
import jax.numpy as jnp
from jax import lax
from jax.experimental import pallas as pl
from jax.experimental.pallas import tpu as pltpu
from jax.experimental.pallas import tpu_sc as plsc

_B = 4
_S = 4096
_D = 1024
_NC = 2
_NS = 16
_NW = _NC * _NS
_ROWS_PER_W = _S // _NW
_CH = 32
_NCHUNK = _ROWS_PER_W // _CH

_mesh = plsc.VectorSubcoreMesh(core_axis_name="c", subcore_axis_name="s")


_ROWS_PER_CORE = _S // _NC
_NISS = 8
_ICH = 64
_IROWS = _ROWS_PER_CORE // _NISS
_INCHUNK = _IROWS // _ICH


@functools.partial(
    pl.kernel,
    mesh=_mesh,
    out_type=jax.ShapeDtypeStruct((_B, _S, _D), jnp.float32),
    scratch_types=[
        pltpu.VMEM_SHARED((_NISS, 2, _ICH, _D), jnp.float32),
        pltpu.SemaphoreType.DMA((2,)),
        pltpu.SemaphoreType.DMA((2,)),
    ],
)
def _pe_spmem_only(table_hbm, out_hbm, sbuf, lsem, ssem):
    c = lax.axis_index("c")
    s = lax.axis_index("s")
    core_base = c * _ROWS_PER_CORE

    @pl.when(s < _NISS)
    def _issue():
        base = core_base + s * _IROWS

        def load(i):
            return pltpu.async_copy(
                table_hbm.at[pl.ds(base + i * _ICH, _ICH)],
                sbuf.at[s, i % 2], lsem.at[i % 2])

        def store(i, b):
            return pltpu.async_copy(
                sbuf.at[s, i % 2],
                out_hbm.at[b, pl.ds(base + i * _ICH, _ICH)],
                ssem.at[i % 2])

        lh = [None] * _INCHUNK
        sh = [None] * _INCHUNK
        lh[0], lh[1] = load(0), load(1)
        for i in range(_INCHUNK):
            if i > 0 and i + 1 < _INCHUNK:
                for h in sh[i - 1]:
                    h.wait()
                lh[i + 1] = load(i + 1)
            lh[i].wait()
            sh[i] = [store(i, b) for b in range(_B)]
        for i in range(max(0, _INCHUNK - 2), _INCHUNK):
            for h in sh[i]:
                h.wait()


_TROWS = 64
_TCH = 32
_SROWS = _ROWS_PER_CORE - _NS * _TROWS
_SCH = 512


@functools.partial(
    pl.kernel,
    mesh=_mesh,
    out_type=jax.ShapeDtypeStruct((_B, _S, _D), jnp.float32),
    scratch_types=[
        pltpu.VMEM((2, _TCH, _D), jnp.float32),
        pltpu.SemaphoreType.DMA((2,)),
        pltpu.SemaphoreType.DMA((2,)),
        pltpu.VMEM_SHARED((2, _SCH, _D), jnp.float32),
        pltpu.SemaphoreType.DMA((2,)),
        pltpu.SemaphoreType.DMA((2,)),
    ],
)
def _pe_dual(table_hbm, out_hbm, buf, lsem, ssem, sbuf, slsem, sssem):
    c = lax.axis_index("c")
    s = lax.axis_index("s")
    core_base = c * _ROWS_PER_CORE
    tile_base = core_base + s * _TROWS

    def tload(i):
        return pltpu.async_copy(
            table_hbm.at[pl.ds(tile_base + i * _TCH, _TCH)],
            buf.at[i], lsem.at[i])

    def tstore(i, b):
        return pltpu.async_copy(
            buf.at[i],
            out_hbm.at[b, pl.ds(tile_base + i * _TCH, _TCH)],
            ssem.at[i])

    lh = [tload(0), tload(1)]
    sh = []
    for i in range(2):
        lh[i].wait()
        sh += [tstore(i, b) for b in range(_B)]

    spmem_base = core_base + _NS * _TROWS

    @pl.when(s == 0)
    def _spmem_path():
        def sload(i):
            return pltpu.async_copy(
                table_hbm.at[pl.ds(spmem_base + i * _SCH, _SCH)],
                sbuf.at[i], slsem.at[i])

        def sstore(i, b):
            return pltpu.async_copy(
                sbuf.at[i],
                out_hbm.at[b, pl.ds(spmem_base + i * _SCH, _SCH)],
                sssem.at[i])

        slh = [sload(0), sload(1)]
        ssh = []
        for i in range(2):
            slh[i].wait()
            ssh += [sstore(i, b) for b in range(_B)]
        for h in ssh:
            h.wait()

    for h in sh:
        h.wait()


_D_TROWS = 64
_D_TCH = 32
_D_NISS = 8
_D_IROWS = 128
_D_SCH = 64
assert 16 * _D_TROWS + _D_NISS * _D_IROWS == _ROWS_PER_CORE


@functools.partial(
    pl.kernel,
    mesh=_mesh,
    out_type=jax.ShapeDtypeStruct((_B, _S, _D), jnp.float32),
    scratch_types=[
        pltpu.VMEM((2, _D_TCH, _D), jnp.float32),
        pltpu.SemaphoreType.DMA((2,)),
        pltpu.SemaphoreType.DMA((2,)),
        pltpu.VMEM_SHARED((_D_NISS, 2, _D_SCH, _D), jnp.float32),
        pltpu.SemaphoreType.DMA((2,)),
        pltpu.SemaphoreType.DMA((2,)),
    ],
)
def _pe_dual2(table_hbm, out_hbm, tbuf, tlsem, tssem, sbuf, slsem, sssem):
    c = lax.axis_index("c")
    s = lax.axis_index("s")
    core_base = c * _ROWS_PER_CORE
    tile_base = core_base + s * _D_TROWS
    spmem_base = core_base + 16 * _D_TROWS

    def sload_desc(i):
        ibase = spmem_base + s * _D_IROWS
        return pltpu.make_async_copy(
            table_hbm.at[pl.ds(ibase + i * _D_SCH, _D_SCH)],
            sbuf.at[s, i], slsem.at[i])

    @pl.when(s < _D_NISS)
    def _prime():
        sload_desc(0).start()
        sload_desc(1).start()

    tl = [
        pltpu.async_copy(
            table_hbm.at[pl.ds(tile_base + i * _D_TCH, _D_TCH)],
            tbuf.at[i], tlsem.at[i])
        for i in range(2)
    ]
    th = []
    for i in range(2):
        tl[i].wait()
        th += [
            pltpu.async_copy(
                tbuf.at[i],
                out_hbm.at[b, pl.ds(tile_base + i * _D_TCH, _D_TCH)],
                tssem.at[i])
            for b in range(_B)
        ]

    @pl.when(s < _D_NISS)
    def _spmem_stores():
        ibase = spmem_base + s * _D_IROWS
        sh = []
        for i in range(2):
            sload_desc(i).wait()
            sh += [
                pltpu.async_copy(
                    sbuf.at[s, i],
                    out_hbm.at[b, pl.ds(ibase + i * _D_SCH, _D_SCH)],
                    sssem.at[i])
                for b in range(_B)
            ]
        for h in sh:
            h.wait()

    for h in th:
        h.wait()


_CH3 = 32
_NSLOT = 3
_NCHUNK3 = _ROWS_PER_W // _CH3


@functools.partial(
    pl.kernel,
    mesh=_mesh,
    out_type=jax.ShapeDtypeStruct((_B, _S, _D), jnp.float32),
    scratch_types=[
        pltpu.VMEM((_NSLOT, _CH3, _D), jnp.float32),
        pltpu.SemaphoreType.DMA((_NSLOT,)),
        pltpu.SemaphoreType.DMA((_NSLOT,)),
    ],
)
def _pe_ring(table_hbm, out_hbm, buf, load_sem, store_sem):
    wid = lax.axis_index("s") * _NC + lax.axis_index("c")
    base = wid * _ROWS_PER_W

    def load(i):
        return pltpu.async_copy(
            table_hbm.at[pl.ds(base + i * _CH3, _CH3)],
            buf.at[i % _NSLOT],
            load_sem.at[i % _NSLOT])

    def store(i, b):
        return pltpu.async_copy(
            buf.at[i % _NSLOT],
            out_hbm.at[b, pl.ds(base + i * _CH3, _CH3)],
            store_sem.at[i % _NSLOT])

    load_h = [None] * _NCHUNK3
    store_h = [None] * _NCHUNK3
    for j in range(min(_NSLOT, _NCHUNK3)):
        load_h[j] = load(j)
    for i in range(_NCHUNK3):
        if i > 0 and i - 1 + _NSLOT < _NCHUNK3:
            for sh in store_h[i - 1]:
                sh.wait()
            load_h[i - 1 + _NSLOT] = load(i - 1 + _NSLOT)
        load_h[i].wait()
        store_h[i] = [store(i, b) for b in range(_B)]
    waited = set(range(0, max(0, _NCHUNK3 - _NSLOT)))
    for i in range(_NCHUNK3):
        if i not in waited:
            for sh in store_h[i]:
                sh.wait()


@functools.partial(
    pl.kernel,
    mesh=_mesh,
    out_type=jax.ShapeDtypeStruct((_B, _S, _D), jnp.float32),
    scratch_types=[
        pltpu.VMEM((2, _CH, _D), jnp.float32),
        pltpu.SemaphoreType.DMA((2,)),
        pltpu.SemaphoreType.DMA((2,)),
    ],
)
def _pe_broadcast(table_hbm, out_hbm, buf, load_sem, store_sem):
    wid = lax.axis_index("s") * _NC + lax.axis_index("c")
    base = wid * _ROWS_PER_W

    def load(i, slot):
        return pltpu.async_copy(
            table_hbm.at[pl.ds(base + i * _CH, _CH)],
            buf.at[slot],
            load_sem.at[slot])

    def store(i, slot, b):
        return pltpu.async_copy(
            buf.at[slot],
            out_hbm.at[b, pl.ds(base + i * _CH, _CH)],
            store_sem.at[slot])

    pending_stores = {0: [], 1: []}
    h = load(0, 0)
    for i in range(_NCHUNK):
        s = i % 2
        if i + 1 < _NCHUNK:
            ns = 1 - s
            for sh in pending_stores[ns]:
                sh.wait()
            pending_stores[ns] = []
            next_h = load(i + 1, ns)
        h.wait()
        pending_stores[s] = [store(i, s, b) for b in range(_B)]
        if i + 1 < _NCHUNK:
            h = next_h
    for s in (0, 1):
        for sh in pending_stores[s]:
            sh.wait()


_TRI_CHUNKS = ((0, 48, 0), (48, 48, 1), (96, 32, 0))


@functools.partial(
    pl.kernel,
    mesh=_mesh,
    out_type=jax.ShapeDtypeStruct((_B, _S, _D), jnp.float32),
    scratch_types=[
        pltpu.VMEM((2, 48, _D), jnp.float32),
        pltpu.SemaphoreType.DMA((2,)),
        pltpu.SemaphoreType.DMA((2,)),
    ],
)
def _pe_tri(table_hbm, out_hbm, buf, lsem, ssem):
    wid = lax.axis_index("s") * _NC + lax.axis_index("c")
    base = wid * _ROWS_PER_W

    def load(i):
        off, n, slot = _TRI_CHUNKS[i]
        return pltpu.async_copy(
            table_hbm.at[pl.ds(base + off, n)],
            buf.at[slot, pl.ds(0, n)], lsem.at[slot])

    def store(i, b):
        off, n, slot = _TRI_CHUNKS[i]
        return pltpu.async_copy(
            buf.at[slot, pl.ds(0, n)],
            out_hbm.at[b, pl.ds(base + off, n)], ssem.at[slot])

    lh0, lh1 = load(0), load(1)
    lh0.wait()
    sh0 = [store(0, b) for b in range(_B)]
    lh1.wait()
    sh1 = [store(1, b) for b in range(_B)]
    for h in sh0:
        h.wait()
    lh2 = load(2)
    lh2.wait()
    sh2 = [store(2, b) for b in range(_B)]
    for h in sh1 + sh2:
        h.wait()


def kernel(inputs, position_embeddings):
    del inputs
    return _pe_tri(position_embeddings)


# speedup vs baseline: 1.2139x; 1.0035x over previous
import functools

import jax
import jax.numpy as jnp
from jax import lax
from jax.experimental import pallas as pl
from jax.experimental.pallas import tpu as pltpu
from jax.experimental.pallas import tpu_sc as plsc

_B = 4
_S = 4096
_D = 1024
_NC = 2
_NS = 16
_NW = _NC * _NS
_ROWS_PER_W = _S // _NW
_CH = 32
_NSLOT = 3
_NCHUNK = _ROWS_PER_W // _CH

_mesh = plsc.VectorSubcoreMesh(core_axis_name="c", subcore_axis_name="s")


@functools.partial(
    pl.kernel,
    mesh=_mesh,
    out_type=jax.ShapeDtypeStruct((_B, _S, _D), jnp.float32),
    scratch_types=[
        pltpu.VMEM((_NSLOT, _CH, _D), jnp.float32),
        pltpu.SemaphoreType.DMA((_NSLOT,)),
        pltpu.SemaphoreType.DMA((_NSLOT,)),
    ],
)
def _pe_ring(table_hbm, out_hbm, buf, load_sem, store_sem):
    wid = lax.axis_index("s") * _NC + lax.axis_index("c")
    base = wid * _ROWS_PER_W

    def load(i):
        return pltpu.async_copy(
            table_hbm.at[pl.ds(base + i * _CH, _CH)],
            buf.at[i % _NSLOT],
            load_sem.at[i % _NSLOT])

    def store(i, b):
        return pltpu.async_copy(
            buf.at[i % _NSLOT],
            out_hbm.at[b, pl.ds(base + i * _CH, _CH)],
            store_sem.at[i % _NSLOT])

    load_h = [None] * _NCHUNK
    store_h = [None] * _NCHUNK
    for j in range(min(_NSLOT, _NCHUNK)):
        load_h[j] = load(j)
    for i in range(_NCHUNK):
        if i > 0 and i - 1 + _NSLOT < _NCHUNK:
            for sh in store_h[i - 1]:
                sh.wait()
            load_h[i - 1 + _NSLOT] = load(i - 1 + _NSLOT)
        load_h[i].wait()
        store_h[i] = [store(i, b) for b in range(_B)]
    waited = set(range(0, max(0, _NCHUNK - _NSLOT)))
    for i in range(_NCHUNK):
        if i not in waited:
            for sh in store_h[i]:
                sh.wait()


def kernel(inputs, position_embeddings):
    del inputs
    return _pe_ring(position_embeddings)
